# Initial kernel scaffold; baseline (speedup 1.0000x reference)
import jax
import jax.numpy as jnp
from jax.experimental import pallas as pl


def kernel(x, edge_index, eW0, eb0, eWg0, ebg0, eW1, eb1, eWg1, ebg1, nW, nb, nWg, nbg, lW, lb):
    raise NotImplementedError("write your pallas kernel here")


# speedup vs baseline: 4.2209x; 4.2209x over previous
import functools

import jax
import jax.numpy as jnp
import numpy as np
from jax import lax
from jax.experimental import pallas as pl
from jax.experimental.pallas import tpu as pltpu
from jax.experimental.pallas import tpu_sc as plsc

SQ3 = float(np.sqrt(3.0))
SQ15 = float(np.sqrt(15.0))
SQ5 = float(np.sqrt(5.0))

NC = 2
NS = 16
NW = NC * NS

TW = 128
OW = 128

NBLK = 1000
EBLK = 2000


def _proj_body(x_ref, ws_ref, wd_ref, s_ref, d_ref):
    xb = x_ref[...]
    xyz = xb[:, 0:3]
    zpad = jnp.zeros((xb.shape[0], TW - 67), jnp.float32)
    ps = jnp.dot(xb, ws_ref[...], preferred_element_type=jnp.float32)
    pd = jnp.dot(xb, wd_ref[...], preferred_element_type=jnp.float32)
    s_ref[...] = jnp.concatenate([ps, xyz, zpad], axis=1)
    d_ref[...] = jnp.concatenate([pd, -xyz, zpad], axis=1)


def _tc_proj(nodes, ws_cat, wd_cat):
    n = nodes.shape[0]
    grid = n // NBLK
    return pl.pallas_call(
        _proj_body,
        grid=(grid,),
        in_specs=[
            pl.BlockSpec((NBLK, 128), lambda i: (i, 0)),
            pl.BlockSpec((128, 64), lambda i: (0, 0)),
            pl.BlockSpec((128, 64), lambda i: (0, 0)),
        ],
        out_specs=[
            pl.BlockSpec((NBLK, TW), lambda i: (i, 0)),
            pl.BlockSpec((NBLK, TW), lambda i: (i, 0)),
        ],
        out_shape=[
            jax.ShapeDtypeStruct((n, TW), jnp.float32),
            jax.ShapeDtypeStruct((n, TW), jnp.float32),
        ],
        compiler_params=pltpu.CompilerParams(dimension_semantics=("parallel",)),
    )(nodes, ws_cat, wd_cat)


def _edge_body(s_ref, d_ref, w0a_ref, w1m_ref, w1a_ref, o_ref):
    rs = s_ref[...] + d_ref[...]
    r = rs[:, 64:67]
    n2 = jnp.sum(r * r, axis=-1, keepdims=True) + 1e-12
    u = r / jnp.sqrt(n2)
    ux, uy, uz = u[:, 0:1], u[:, 1:2], u[:, 2:3]
    b = rs.shape[0]
    one = jnp.ones((b, 1), jnp.float32)
    a16 = jnp.concatenate([
        one, SQ3 * ux, SQ3 * uy, SQ3 * uz,
        SQ15 * ux * uy, SQ15 * uy * uz, (SQ5 / 2.0) * (3.0 * uz * uz - 1.0),
        SQ15 * ux * uz, (SQ15 / 2.0) * (ux * ux - uy * uy),
        one, jnp.zeros((b, 6), jnp.float32)], axis=1)
    z = rs[:, :64] + jnp.dot(a16, w0a_ref[...], preferred_element_type=jnp.float32)
    m0 = z[:, :32] * jax.nn.sigmoid(z[:, 32:64])
    z1 = (jnp.dot(m0, w1m_ref[...], preferred_element_type=jnp.float32)
          + jnp.dot(a16, w1a_ref[...], preferred_element_type=jnp.float32))
    m1 = z1[:, :32] * jax.nn.sigmoid(z1[:, 32:64])
    o_ref[...] = jnp.concatenate(
        [m1, a16, jnp.zeros((b, OW - 48), jnp.float32)], axis=1)


def _tc_edge(srows, drows, w0a, w1m, w1a):
    e = srows.shape[0]
    grid = e // EBLK
    return pl.pallas_call(
        _edge_body,
        grid=(grid,),
        in_specs=[
            pl.BlockSpec((EBLK, TW), lambda i: (i, 0)),
            pl.BlockSpec((EBLK, TW), lambda i: (i, 0)),
            pl.BlockSpec((16, 64), lambda i: (0, 0)),
            pl.BlockSpec((32, 64), lambda i: (0, 0)),
            pl.BlockSpec((16, 64), lambda i: (0, 0)),
        ],
        out_specs=pl.BlockSpec((EBLK, OW), lambda i: (i, 0)),
        out_shape=jax.ShapeDtypeStruct((e, OW), jnp.float32),
        compiler_params=pltpu.CompilerParams(dimension_semantics=("parallel",)),
    )(srows, drows, w0a, w1m, w1a)


def _node_body(p0_ref, p1_ref, x_ref, wh_ref, wm_ref, bc_ref, lw_ref, lb_ref, o_ref):
    acc = p0_ref[...] + p1_ref[...]
    cnt = acc[:, 41:42]
    inv = 1.0 / jnp.maximum(cnt, 1.0)
    ms = acc * inv
    xb = x_ref[...]
    h = xb
    for l in range(2):
        p = (jnp.dot(h, wh_ref[l], preferred_element_type=jnp.float32)
             + jnp.dot(ms, wm_ref[l], preferred_element_type=jnp.float32)
             + bc_ref[l])
        h = p[:, :128] * jax.nn.sigmoid(p[:, 128:])
    o_ref[...] = (jnp.dot(h, lw_ref[...], preferred_element_type=jnp.float32)
                  + lb_ref[...] + xb)


def _tc_node(p0, p1, nodes, wh, wm, bc, lw, lb):
    n = nodes.shape[0]
    grid = n // NBLK
    return pl.pallas_call(
        _node_body,
        grid=(grid,),
        in_specs=[
            pl.BlockSpec((NBLK, OW), lambda i: (i, 0)),
            pl.BlockSpec((NBLK, OW), lambda i: (i, 0)),
            pl.BlockSpec((NBLK, 128), lambda i: (i, 0)),
            pl.BlockSpec((2, 128, 256), lambda i: (0, 0, 0)),
            pl.BlockSpec((2, OW, 256), lambda i: (0, 0, 0)),
            pl.BlockSpec((2, 1, 256), lambda i: (0, 0, 0)),
            pl.BlockSpec((128, 128), lambda i: (0, 0)),
            pl.BlockSpec((1, 128), lambda i: (0, 0)),
        ],
        out_specs=pl.BlockSpec((NBLK, 128), lambda i: (i, 0)),
        out_shape=jax.ShapeDtypeStruct((n, 128), jnp.float32),
        compiler_params=pltpu.CompilerParams(dimension_semantics=("parallel",)),
    )(p0, p1, nodes, wh, wm, bc, lw, lb)


def _sc_gather(stab, dtab, sidx3, didx3, e, ch, nch, epw):
    mesh = plsc.VectorSubcoreMesh(core_axis_name="c", subcore_axis_name="s",
                                  num_cores=NC, num_subcores=NS)

    @functools.partial(
        pl.kernel,
        out_type=(jax.ShapeDtypeStruct((e, TW), jnp.float32),
                  jax.ShapeDtypeStruct((e, TW), jnp.float32)),
        mesh=mesh,
        scratch_types=[
            pltpu.VMEM((nch, ch), jnp.int32),
            pltpu.VMEM((nch, ch), jnp.int32),
            pltpu.VMEM((ch, TW), jnp.float32),
            pltpu.VMEM((ch, TW), jnp.float32),
            pltpu.SemaphoreType.DMA,
            pltpu.SemaphoreType.DMA,
        ],
    )
    def k(stab_h, dtab_h, sidx_h, didx_h, sout_h, dout_h,
          sidx_v, didx_v, sbuf, dbuf, sem_s, sem_d):
        wid = lax.axis_index("s") * NC + lax.axis_index("c")
        base = wid * epw
        pltpu.sync_copy(sidx_h.at[wid], sidx_v)
        pltpu.sync_copy(didx_h.at[wid], didx_v)

        def body(j, carry):
            cs = pltpu.async_copy(stab_h.at[sidx_v.at[j]], sbuf, sem_s)
            cd = pltpu.async_copy(dtab_h.at[didx_v.at[j]], dbuf, sem_d)
            cs.wait()
            pltpu.sync_copy(sbuf, sout_h.at[pl.ds(base + j * ch, ch)])
            cd.wait()
            pltpu.sync_copy(dbuf, dout_h.at[pl.ds(base + j * ch, ch)])
            return carry

        lax.fori_loop(0, nch, body, 0)

    return k(stab, dtab, sidx3, didx3)


def _sc_scatter(vals, didx3, zrows, n_pad, ch, nch, epw):
    npt = n_pad // NS
    mesh = plsc.VectorSubcoreMesh(core_axis_name="c", subcore_axis_name="s",
                                  num_cores=NC, num_subcores=NS)

    @functools.partial(
        pl.kernel,
        out_type=jax.ShapeDtypeStruct((NC, n_pad, OW), jnp.float32),
        mesh=mesh,
        scratch_types=[
            pltpu.VMEM((nch, ch), jnp.int32),
            pltpu.VMEM((ch, OW), jnp.float32),
            pltpu.VMEM_SHARED((n_pad, OW), jnp.float32),
        ],
    )
    def k(vals_h, didx_h, z_h, out_h, idx_v, buf, acc_sh):
        c = lax.axis_index("c")
        s = lax.axis_index("s")
        wid = s * NC + c
        base = wid * epw
        pltpu.sync_copy(z_h, acc_sh.at[pl.ds(s * npt, npt)])
        pltpu.sync_copy(didx_h.at[wid], idx_v)
        plsc.subcore_barrier()

        def body(j, carry):
            pltpu.sync_copy(vals_h.at[pl.ds(base + j * ch, ch)], buf)
            pltpu.sync_copy(buf, acc_sh.at[idx_v.at[j]], add=True)
            return carry

        lax.fori_loop(0, nch, body, 0)
        plsc.subcore_barrier()
        pltpu.sync_copy(acc_sh.at[pl.ds(s * npt, npt)], out_h.at[c, pl.ds(s * npt, npt)])

    return k(vals, didx3, zrows)


def kernel(x, edge_index, eW0, eb0, eWg0, ebg0, eW1, eb1, eWg1, ebg1,
           nW, nb, nWg, nbg, lW, lb):
    n, d = x.shape
    e = edge_index.shape[1]
    s_steps = eW0.shape[0]
    h = eW0.shape[2]

    epw = e // NW
    ch = 80
    nch = epw // ch
    n_pad = ((n + 8 * NS - 1) // (8 * NS)) * (8 * NS)

    src = edge_index[0]
    dst = edge_index[1]
    sidx3 = src.reshape(NW, nch, ch)
    didx3 = dst.reshape(NW, nch, ch)
    zrows = jnp.zeros((n_pad // NS, OW), jnp.float32)

    nodes = x
    for s in range(s_steps):
        ws_cat = jnp.concatenate([eW0[s][:d], eWg0[s][:d]], axis=1)
        wd_cat = jnp.concatenate([eW0[s][d:2 * d], eWg0[s][d:2 * d]], axis=1)
        w0a = jnp.zeros((16, 2 * h), jnp.float32)
        w0a = w0a.at[:9, :h].set(eW0[s][2 * d:2 * d + 9]).at[9, :h].set(eb0[s])
        w0a = w0a.at[:9, h:].set(eWg0[s][2 * d:2 * d + 9]).at[9, h:].set(ebg0[s])
        w1m = jnp.concatenate([eW1[s][:h], eWg1[s][:h]], axis=1)
        w1a = jnp.zeros((16, 2 * h), jnp.float32)
        w1a = w1a.at[:9, :h].set(eW1[s][h:h + 9]).at[9, :h].set(eb1[s])
        w1a = w1a.at[:9, h:].set(eWg1[s][h:h + 9]).at[9, h:].set(ebg1[s])
        wh = jnp.stack([jnp.concatenate([nW[s, l][:d], nWg[s, l][:d]], axis=1)
                        for l in range(2)])
        wm = jnp.stack([
            jnp.zeros((OW, 2 * d), jnp.float32)
            .at[:41, :d].set(nW[s, l][d:d + 41])
            .at[:41, d:].set(nWg[s, l][d:d + 41])
            for l in range(2)])
        bc = jnp.stack([jnp.concatenate([nb[s, l], nbg[s, l]])[None, :]
                        for l in range(2)])

        stab, dtab = _tc_proj(nodes, ws_cat, wd_cat)
        srows, drows = _sc_gather(stab, dtab, sidx3, didx3, e, ch, nch, epw)
        evals = _tc_edge(srows, drows, w0a, w1m, w1a)
        partials = _sc_scatter(evals, didx3, zrows, n_pad, ch, nch, epw)
        nodes = _tc_node(partials[0, :n], partials[1, :n], nodes, wh, wm, bc,
                         lW[s], lb[s][None, :])
    return nodes


# speedup vs baseline: 7.2904x; 1.7272x over previous
import functools

import jax
import jax.numpy as jnp
import numpy as np
from jax import lax
from jax.experimental import pallas as pl
from jax.experimental.pallas import tpu as pltpu
from jax.experimental.pallas import tpu_sc as plsc

SQ3 = float(np.sqrt(3.0))
SQ15 = float(np.sqrt(15.0))
SQ5 = float(np.sqrt(5.0))

NC = 2
NS = 16
NW = NC * NS

TW = 128
OW = 128

NBLK = 1000
EBLK = 2000


def _proj_body(x_ref, ws_ref, wd_ref, s_ref, d_ref):
    xb = x_ref[...]
    xyz = xb[:, 0:3]
    zpad = jnp.zeros((xb.shape[0], TW - 67), jnp.float32)
    ps = jnp.dot(xb, ws_ref[...], preferred_element_type=jnp.float32)
    pd = jnp.dot(xb, wd_ref[...], preferred_element_type=jnp.float32)
    s_ref[...] = jnp.concatenate([ps, xyz, zpad], axis=1)
    d_ref[...] = jnp.concatenate([pd, -xyz, zpad], axis=1)


def _tc_proj(nodes, ws_cat, wd_cat):
    n = nodes.shape[0]
    grid = n // NBLK
    return pl.pallas_call(
        _proj_body,
        grid=(grid,),
        in_specs=[
            pl.BlockSpec((NBLK, 128), lambda i: (i, 0)),
            pl.BlockSpec((128, 64), lambda i: (0, 0)),
            pl.BlockSpec((128, 64), lambda i: (0, 0)),
        ],
        out_specs=[
            pl.BlockSpec((NBLK, TW), lambda i: (i, 0)),
            pl.BlockSpec((NBLK, TW), lambda i: (i, 0)),
        ],
        out_shape=[
            jax.ShapeDtypeStruct((n, TW), jnp.float32),
            jax.ShapeDtypeStruct((n, TW), jnp.float32),
        ],
        compiler_params=pltpu.CompilerParams(dimension_semantics=("parallel",)),
    )(nodes, ws_cat, wd_cat)


def _edge_body(rs_ref, wa_ref, w1m_ref, o_ref):
    rs = rs_ref[...]
    b = rs.shape[0]
    xt = rs[:, 64:80].T
    x, y, z = xt[0:1], xt[1:2], xt[2:3]
    n2 = x * x + y * y + z * z + 1e-12
    rinv = jax.lax.rsqrt(n2)
    ux, uy, uz = x * rinv, y * rinv, z * rinv
    one = jnp.ones((1, b), jnp.float32)
    at = jnp.concatenate([
        one, SQ3 * ux, SQ3 * uy, SQ3 * uz,
        SQ15 * ux * uy, SQ15 * uy * uz, (SQ5 / 2.0) * (3.0 * uz * uz - 1.0),
        SQ15 * ux * uz, (SQ15 / 2.0) * (ux * ux - uy * uy),
        one, jnp.zeros((6, b), jnp.float32)], axis=0)
    a16 = at.T
    av = jnp.dot(a16, wa_ref[...], preferred_element_type=jnp.float32)
    z0 = rs[:, :64] + av[:, :64]
    m0 = z0[:, :32] * jax.nn.sigmoid(z0[:, 32:64])
    z1 = (jnp.dot(m0, w1m_ref[...], preferred_element_type=jnp.float32)
          + av[:, 64:128])
    m1 = z1[:, :32] * jax.nn.sigmoid(z1[:, 32:64])
    o_ref[...] = jnp.concatenate(
        [m1, a16, jnp.zeros((b, OW - 48), jnp.float32)], axis=1)


def _tc_edge(rsrows, wa, w1m):
    e = rsrows.shape[0]
    grid = e // EBLK
    return pl.pallas_call(
        _edge_body,
        grid=(grid,),
        in_specs=[
            pl.BlockSpec((EBLK, TW), lambda i: (i, 0)),
            pl.BlockSpec((16, 128), lambda i: (0, 0)),
            pl.BlockSpec((32, 64), lambda i: (0, 0)),
        ],
        out_specs=pl.BlockSpec((EBLK, OW), lambda i: (i, 0)),
        out_shape=jax.ShapeDtypeStruct((e, OW), jnp.float32),
        compiler_params=pltpu.CompilerParams(dimension_semantics=("parallel",)),
    )(rsrows, wa, w1m)


def _node_body(p0_ref, p1_ref, x_ref, wh_ref, wm_ref, bc_ref, lw_ref, lb_ref, o_ref):
    acc = p0_ref[...] + p1_ref[...]
    cnt = acc[:, 41:42]
    inv = 1.0 / jnp.maximum(cnt, 1.0)
    ms = acc * inv
    xb = x_ref[...]
    h = xb
    for l in range(2):
        p = (jnp.dot(h, wh_ref[l], preferred_element_type=jnp.float32)
             + jnp.dot(ms, wm_ref[l], preferred_element_type=jnp.float32)
             + bc_ref[l])
        h = p[:, :128] * jax.nn.sigmoid(p[:, 128:])
    o_ref[...] = (jnp.dot(h, lw_ref[...], preferred_element_type=jnp.float32)
                  + lb_ref[...] + xb)


def _tc_node(p0, p1, nodes, wh, wm, bc, lw, lb):
    n = nodes.shape[0]
    grid = n // NBLK
    return pl.pallas_call(
        _node_body,
        grid=(grid,),
        in_specs=[
            pl.BlockSpec((NBLK, OW), lambda i: (i, 0)),
            pl.BlockSpec((NBLK, OW), lambda i: (i, 0)),
            pl.BlockSpec((NBLK, 128), lambda i: (i, 0)),
            pl.BlockSpec((2, 128, 256), lambda i: (0, 0, 0)),
            pl.BlockSpec((2, OW, 256), lambda i: (0, 0, 0)),
            pl.BlockSpec((2, 1, 256), lambda i: (0, 0, 0)),
            pl.BlockSpec((128, 128), lambda i: (0, 0)),
            pl.BlockSpec((1, 128), lambda i: (0, 0)),
        ],
        out_specs=pl.BlockSpec((NBLK, 128), lambda i: (i, 0)),
        out_shape=jax.ShapeDtypeStruct((n, 128), jnp.float32),
        compiler_params=pltpu.CompilerParams(dimension_semantics=("parallel",)),
    )(p0, p1, nodes, wh, wm, bc, lw, lb)


def _sc_gather(stab, dtab, sidx3, didx3, e, ch, nch, epw):
    mesh = plsc.VectorSubcoreMesh(core_axis_name="c", subcore_axis_name="s",
                                  num_cores=NC, num_subcores=NS)

    @functools.partial(
        pl.kernel,
        out_type=jax.ShapeDtypeStruct((e, TW), jnp.float32),
        mesh=mesh,
        scratch_types=[
            pltpu.VMEM((nch, ch), jnp.int32),
            pltpu.VMEM((nch, ch), jnp.int32),
            pltpu.VMEM((ch, TW), jnp.float32),
            pltpu.VMEM((ch, TW), jnp.float32),
            pltpu.VMEM((ch, TW), jnp.float32),
            pltpu.VMEM((ch, TW), jnp.float32),
            pltpu.SemaphoreType.DMA,
            pltpu.SemaphoreType.DMA,
            pltpu.SemaphoreType.DMA,
            pltpu.SemaphoreType.DMA,
        ],
    )
    def k(stab_h, dtab_h, sidx_h, didx_h, out_h,
          sidx_v, didx_v, sA, dA, sB, dB, semsA, semdA, semsB, semdB):
        wid = lax.axis_index("s") * NC + lax.axis_index("c")
        base = wid * epw
        pltpu.sync_copy(sidx_h.at[wid], sidx_v)
        pltpu.sync_copy(didx_h.at[wid], didx_v)

        def issue(j, sbuf, dbuf, sem_s, sem_d):
            pltpu.async_copy(stab_h.at[sidx_v.at[j]], sbuf, sem_s)
            pltpu.async_copy(dtab_h.at[didx_v.at[j]], dbuf, sem_d)

        def wait(sbuf, dbuf, sem_s, sem_d):
            pltpu.make_async_copy(stab_h.at[sidx_v.at[0]], sbuf, sem_s).wait()
            pltpu.make_async_copy(dtab_h.at[didx_v.at[0]], dbuf, sem_d).wait()

        def process(j, sbuf, dbuf):
            def row(r, carry):
                for g in range(TW // 16):
                    sl = pl.ds(g * 16, 16)
                    sbuf[r, sl] = sbuf[r, sl] + dbuf[r, sl]
                return carry

            lax.fori_loop(0, ch, row, 0)
            pltpu.sync_copy(sbuf, out_h.at[pl.ds(base + j * ch, ch)])

        issue(0, sA, dA, semsA, semdA)

        def body(k2, carry):
            j = 2 * k2
            issue(j + 1, sB, dB, semsB, semdB)
            wait(sA, dA, semsA, semdA)
            process(j, sA, dA)
            issue(j + 2, sA, dA, semsA, semdA)
            wait(sB, dB, semsB, semdB)
            process(j + 1, sB, dB)
            return carry

        lax.fori_loop(0, (nch - 1) // 2, body, 0)
        wait(sA, dA, semsA, semdA)
        process(nch - 1, sA, dA)

    return k(stab, dtab, sidx3, didx3)


def _sc_scatter(vals, didx3, zrows, n_pad, ch, nch, epw):
    npt = n_pad // NS
    mesh = plsc.VectorSubcoreMesh(core_axis_name="c", subcore_axis_name="s",
                                  num_cores=NC, num_subcores=NS)

    @functools.partial(
        pl.kernel,
        out_type=jax.ShapeDtypeStruct((NC, n_pad, OW), jnp.float32),
        mesh=mesh,
        scratch_types=[
            pltpu.VMEM((nch, ch), jnp.int32),
            pltpu.VMEM((ch, OW), jnp.float32),
            pltpu.VMEM((ch, OW), jnp.float32),
            pltpu.SemaphoreType.DMA,
            pltpu.SemaphoreType.DMA,
            pltpu.VMEM_SHARED((n_pad, OW), jnp.float32),
        ],
    )
    def k(vals_h, didx_h, z_h, out_h, idx_v, bufA, bufB, semA, semB, acc_sh):
        c = lax.axis_index("c")
        s = lax.axis_index("s")
        wid = s * NC + c
        base = wid * epw
        pltpu.sync_copy(z_h, acc_sh.at[pl.ds(s * npt, npt)])
        pltpu.sync_copy(didx_h.at[wid], idx_v)
        plsc.subcore_barrier()

        def issue(j, buf, sem):
            pltpu.async_copy(vals_h.at[pl.ds(base + j * ch, ch)], buf, sem)

        def wait(buf, sem):
            pltpu.make_async_copy(vals_h.at[pl.ds(base, ch)], buf, sem).wait()

        def scat(j, buf):
            pltpu.sync_copy(buf, acc_sh.at[idx_v.at[j]], add=True)

        issue(0, bufA, semA)

        def body(k2, carry):
            j = 2 * k2
            issue(j + 1, bufB, semB)
            wait(bufA, semA)
            scat(j, bufA)
            issue(j + 2, bufA, semA)
            wait(bufB, semB)
            scat(j + 1, bufB)
            return carry

        lax.fori_loop(0, (nch - 1) // 2, body, 0)
        wait(bufA, semA)
        scat(nch - 1, bufA)
        plsc.subcore_barrier()
        pltpu.sync_copy(acc_sh.at[pl.ds(s * npt, npt)], out_h.at[c, pl.ds(s * npt, npt)])

    return k(vals, didx3, zrows)


def kernel(x, edge_index, eW0, eb0, eWg0, ebg0, eW1, eb1, eWg1, ebg1,
           nW, nb, nWg, nbg, lW, lb):
    n, d = x.shape
    e = edge_index.shape[1]
    s_steps = eW0.shape[0]
    h = eW0.shape[2]

    epw = e // NW
    ch = 80
    nch = epw // ch
    n_pad = ((n + 8 * NS - 1) // (8 * NS)) * (8 * NS)

    src = edge_index[0]
    dst = edge_index[1]
    sidx3 = src.reshape(NW, nch, ch)
    didx3 = dst.reshape(NW, nch, ch)
    zrows = jnp.zeros((n_pad // NS, OW), jnp.float32)

    nodes = x
    for s in range(s_steps):
        ws_cat = jnp.concatenate([eW0[s][:d], eWg0[s][:d]], axis=1)
        wd_cat = jnp.concatenate([eW0[s][d:2 * d], eWg0[s][d:2 * d]], axis=1)
        wa = jnp.zeros((16, 4 * h), jnp.float32)
        wa = wa.at[:9, :h].set(eW0[s][2 * d:2 * d + 9]).at[9, :h].set(eb0[s])
        wa = wa.at[:9, h:2 * h].set(eWg0[s][2 * d:2 * d + 9]).at[9, h:2 * h].set(ebg0[s])
        wa = wa.at[:9, 2 * h:3 * h].set(eW1[s][h:h + 9]).at[9, 2 * h:3 * h].set(eb1[s])
        wa = wa.at[:9, 3 * h:].set(eWg1[s][h:h + 9]).at[9, 3 * h:].set(ebg1[s])
        w1m = jnp.concatenate([eW1[s][:h], eWg1[s][:h]], axis=1)
        wh = jnp.stack([jnp.concatenate([nW[s, l][:d], nWg[s, l][:d]], axis=1)
                        for l in range(2)])
        wm = jnp.stack([
            jnp.zeros((OW, 2 * d), jnp.float32)
            .at[:41, :d].set(nW[s, l][d:d + 41])
            .at[:41, d:].set(nWg[s, l][d:d + 41])
            for l in range(2)])
        bc = jnp.stack([jnp.concatenate([nb[s, l], nbg[s, l]])[None, :]
                        for l in range(2)])

        stab, dtab = _tc_proj(nodes, ws_cat, wd_cat)
        rsrows = _sc_gather(stab, dtab, sidx3, didx3, e, ch, nch, epw)
        evals = _tc_edge(rsrows, wa, w1m)
        partials = _sc_scatter(evals, didx3, zrows, n_pad, ch, nch, epw)
        nodes = _tc_node(partials[0, :n], partials[1, :n], nodes, wh, wm, bc,
                         lW[s], lb[s][None, :])
    return nodes


# speedup vs baseline: 8.1701x; 1.1207x over previous
import functools

import jax
import jax.numpy as jnp
import numpy as np
from jax import lax
from jax.experimental import pallas as pl
from jax.experimental.pallas import tpu as pltpu
from jax.experimental.pallas import tpu_sc as plsc

SQ3 = float(np.sqrt(3.0))
SQ15 = float(np.sqrt(15.0))
SQ5 = float(np.sqrt(5.0))

NC = 2
NS = 16
NW = NC * NS

TW = 128
OW = 128

NBLK = 1000
EBLK = 2000


def _proj_body(x_ref, ws_ref, wd_ref, s_ref, d_ref):
    xb = x_ref[...]
    xyz = xb[:, 0:3]
    zpad = jnp.zeros((xb.shape[0], TW - 67), jnp.float32)
    ps = jnp.dot(xb, ws_ref[...], preferred_element_type=jnp.float32)
    pd = jnp.dot(xb, wd_ref[...], preferred_element_type=jnp.float32)
    s_ref[...] = jnp.concatenate([ps, xyz, zpad], axis=1)
    d_ref[...] = jnp.concatenate([pd, -xyz, zpad], axis=1)


def _tc_proj(nodes, ws_cat, wd_cat):
    n = nodes.shape[0]
    grid = n // NBLK
    return pl.pallas_call(
        _proj_body,
        grid=(grid,),
        in_specs=[
            pl.BlockSpec((NBLK, 128), lambda i: (i, 0)),
            pl.BlockSpec((128, 64), lambda i: (0, 0)),
            pl.BlockSpec((128, 64), lambda i: (0, 0)),
        ],
        out_specs=[
            pl.BlockSpec((NBLK, TW), lambda i: (i, 0)),
            pl.BlockSpec((NBLK, TW), lambda i: (i, 0)),
        ],
        out_shape=[
            jax.ShapeDtypeStruct((n, TW), jnp.float32),
            jax.ShapeDtypeStruct((n, TW), jnp.float32),
        ],
        compiler_params=pltpu.CompilerParams(dimension_semantics=("parallel",)),
    )(nodes, ws_cat, wd_cat)


def _edge_body(rs_ref, wa_ref, w1m_ref, o_ref):
    rs = rs_ref[...]
    b = rs.shape[0]
    xt = rs[:, 64:80].T
    x, y, z = xt[0:1], xt[1:2], xt[2:3]
    n2 = x * x + y * y + z * z + 1e-12
    rinv = jax.lax.rsqrt(n2)
    ux, uy, uz = x * rinv, y * rinv, z * rinv
    one = jnp.ones((1, b), jnp.float32)
    at = jnp.concatenate([
        one, SQ3 * ux, SQ3 * uy, SQ3 * uz,
        SQ15 * ux * uy, SQ15 * uy * uz, (SQ5 / 2.0) * (3.0 * uz * uz - 1.0),
        SQ15 * ux * uz, (SQ15 / 2.0) * (ux * ux - uy * uy),
        one, jnp.zeros((6, b), jnp.float32)], axis=0)
    a16 = at.T
    av = jnp.dot(a16, wa_ref[...], preferred_element_type=jnp.float32)
    z0 = rs[:, :64] + av[:, :64]
    m0 = z0[:, :32] * jax.nn.sigmoid(z0[:, 32:64])
    z1 = (jnp.dot(m0, w1m_ref[...], preferred_element_type=jnp.float32)
          + av[:, 64:128])
    m1 = z1[:, :32] * jax.nn.sigmoid(z1[:, 32:64])
    o_ref[...] = jnp.concatenate(
        [m1, a16, jnp.zeros((b, OW - 48), jnp.float32)], axis=1)


def _tc_edge(rsrows, wa, w1m):
    e = rsrows.shape[0]
    grid = e // EBLK
    return pl.pallas_call(
        _edge_body,
        grid=(grid,),
        in_specs=[
            pl.BlockSpec((EBLK, TW), lambda i: (i, 0)),
            pl.BlockSpec((16, 128), lambda i: (0, 0)),
            pl.BlockSpec((32, 64), lambda i: (0, 0)),
        ],
        out_specs=pl.BlockSpec((EBLK, OW), lambda i: (i, 0)),
        out_shape=jax.ShapeDtypeStruct((e, OW), jnp.float32),
        compiler_params=pltpu.CompilerParams(dimension_semantics=("parallel",)),
    )(rsrows, wa, w1m)


def _node_body(p0_ref, p1_ref, p2_ref, p3_ref, x_ref, wh_ref, wm_ref, bc_ref,
               lw_ref, lb_ref, o_ref):
    acc = (p0_ref[...] + p1_ref[...]) + (p2_ref[...] + p3_ref[...])
    cnt = acc[:, 41:42]
    inv = 1.0 / jnp.maximum(cnt, 1.0)
    ms = acc * inv
    xb = x_ref[...]
    h = xb
    for l in range(2):
        p = (jnp.dot(h, wh_ref[l], preferred_element_type=jnp.float32)
             + jnp.dot(ms, wm_ref[l], preferred_element_type=jnp.float32)
             + bc_ref[l])
        h = p[:, :128] * jax.nn.sigmoid(p[:, 128:])
    o_ref[...] = (jnp.dot(h, lw_ref[...], preferred_element_type=jnp.float32)
                  + lb_ref[...] + xb)


def _tc_node(p0, p1, p2, p3, nodes, wh, wm, bc, lw, lb):
    n = nodes.shape[0]
    grid = n // NBLK
    return pl.pallas_call(
        _node_body,
        grid=(grid,),
        in_specs=[
            pl.BlockSpec((NBLK, OW), lambda i: (i, 0)),
            pl.BlockSpec((NBLK, OW), lambda i: (i, 0)),
            pl.BlockSpec((NBLK, OW), lambda i: (i, 0)),
            pl.BlockSpec((NBLK, OW), lambda i: (i, 0)),
            pl.BlockSpec((NBLK, 128), lambda i: (i, 0)),
            pl.BlockSpec((2, 128, 256), lambda i: (0, 0, 0)),
            pl.BlockSpec((2, OW, 256), lambda i: (0, 0, 0)),
            pl.BlockSpec((2, 1, 256), lambda i: (0, 0, 0)),
            pl.BlockSpec((128, 128), lambda i: (0, 0)),
            pl.BlockSpec((1, 128), lambda i: (0, 0)),
        ],
        out_specs=pl.BlockSpec((NBLK, 128), lambda i: (i, 0)),
        out_shape=jax.ShapeDtypeStruct((n, 128), jnp.float32),
        compiler_params=pltpu.CompilerParams(dimension_semantics=("parallel",)),
    )(p0, p1, p2, p3, nodes, wh, wm, bc, lw, lb)


def _sc_gather(stab, dtab, sidx3, didx3, e, ch, nch, epw):
    mesh = plsc.VectorSubcoreMesh(core_axis_name="c", subcore_axis_name="s",
                                  num_cores=NC, num_subcores=NS)

    @functools.partial(
        pl.kernel,
        out_type=jax.ShapeDtypeStruct((e, TW), jnp.float32),
        mesh=mesh,
        scratch_types=[
            pltpu.VMEM((nch, ch), jnp.int32),
            pltpu.VMEM((nch, ch), jnp.int32),
            pltpu.VMEM((ch, TW), jnp.float32),
            pltpu.VMEM((ch, TW), jnp.float32),
            pltpu.VMEM((ch, TW), jnp.float32),
            pltpu.VMEM((ch, TW), jnp.float32),
            pltpu.SemaphoreType.DMA,
            pltpu.SemaphoreType.DMA,
            pltpu.SemaphoreType.DMA,
            pltpu.SemaphoreType.DMA,
        ],
    )
    def k(stab_h, dtab_h, sidx_h, didx_h, out_h,
          sidx_v, didx_v, sA, dA, sB, dB, semsA, semdA, semsB, semdB):
        wid = lax.axis_index("s") * NC + lax.axis_index("c")
        base = wid * epw
        pltpu.sync_copy(sidx_h.at[wid], sidx_v)
        pltpu.sync_copy(didx_h.at[wid], didx_v)

        def issue(j, sbuf, dbuf, sem_s, sem_d):
            pltpu.async_copy(stab_h.at[sidx_v.at[j]], sbuf, sem_s)
            pltpu.async_copy(dtab_h.at[didx_v.at[j]], dbuf, sem_d)

        def wait(sbuf, dbuf, sem_s, sem_d):
            pltpu.make_async_copy(stab_h.at[sidx_v.at[0]], sbuf, sem_s).wait()
            pltpu.make_async_copy(dtab_h.at[didx_v.at[0]], dbuf, sem_d).wait()

        def process(j, sbuf, dbuf):
            def row(r, carry):
                for g in range(TW // 16):
                    sl = pl.ds(g * 16, 16)
                    sbuf[r, sl] = sbuf[r, sl] + dbuf[r, sl]
                return carry

            lax.fori_loop(0, ch, row, 0)
            pltpu.sync_copy(sbuf, out_h.at[pl.ds(base + j * ch, ch)])

        issue(0, sA, dA, semsA, semdA)

        def body(k2, carry):
            j = 2 * k2
            issue(j + 1, sB, dB, semsB, semdB)
            wait(sA, dA, semsA, semdA)
            process(j, sA, dA)
            issue(j + 2, sA, dA, semsA, semdA)
            wait(sB, dB, semsB, semdB)
            process(j + 1, sB, dB)
            return carry

        lax.fori_loop(0, (nch - 1) // 2, body, 0)
        wait(sA, dA, semsA, semdA)
        process(nch - 1, sA, dA)

    return k(stab, dtab, sidx3, didx3)


def _sc_scatter(vals, didx3, zrows, n_pad, ch, nch, epw):
    npt = n_pad // NS
    mesh = plsc.VectorSubcoreMesh(core_axis_name="c", subcore_axis_name="s",
                                  num_cores=NC, num_subcores=NS)

    @functools.partial(
        pl.kernel,
        out_type=jax.ShapeDtypeStruct((NC, n_pad, OW), jnp.float32),
        mesh=mesh,
        scratch_types=[
            pltpu.VMEM((nch, ch), jnp.int32),
            pltpu.VMEM((ch, OW), jnp.float32),
            pltpu.VMEM((ch, OW), jnp.float32),
            pltpu.SemaphoreType.DMA,
            pltpu.SemaphoreType.DMA,
            pltpu.VMEM_SHARED((n_pad, OW), jnp.float32),
        ],
    )
    def k(vals_h, didx_h, z_h, out_h, idx_v, bufA, bufB, semA, semB, acc_sh):
        c = lax.axis_index("c")
        s = lax.axis_index("s")
        wid = s * NC + c
        base = wid * epw
        pltpu.sync_copy(z_h, acc_sh.at[pl.ds(s * npt, npt)])
        pltpu.sync_copy(didx_h.at[wid], idx_v)
        plsc.subcore_barrier()

        def issue(j, buf, sem):
            pltpu.async_copy(vals_h.at[pl.ds(base + j * ch, ch)], buf, sem)

        def wait(buf, sem):
            pltpu.make_async_copy(vals_h.at[pl.ds(base, ch)], buf, sem).wait()

        def scat(j, buf):
            pltpu.sync_copy(buf, acc_sh.at[idx_v.at[j]], add=True)

        issue(0, bufA, semA)

        def body(k2, carry):
            j = 2 * k2
            issue(j + 1, bufB, semB)
            wait(bufA, semA)
            scat(j, bufA)
            issue(j + 2, bufA, semA)
            wait(bufB, semB)
            scat(j + 1, bufB)
            return carry

        lax.fori_loop(0, (nch - 1) // 2, body, 0)
        wait(bufA, semA)
        scat(nch - 1, bufA)
        plsc.subcore_barrier()
        pltpu.sync_copy(acc_sh.at[pl.ds(s * npt, npt)], out_h.at[c, pl.ds(s * npt, npt)])

    return k(vals, didx3, zrows)


def kernel(x, edge_index, eW0, eb0, eWg0, ebg0, eW1, eb1, eWg1, ebg1,
           nW, nb, nWg, nbg, lW, lb):
    n, d = x.shape
    e = edge_index.shape[1]
    s_steps = eW0.shape[0]
    h = eW0.shape[2]

    e2 = e // 2
    epw = e2 // NW
    ch = 40
    nch = epw // ch
    n_pad = ((n + 8 * NS - 1) // (8 * NS)) * (8 * NS)

    src = edge_index[0]
    dst = edge_index[1]
    sidx = [src[h * e2:(h + 1) * e2].reshape(NW, nch, ch) for h in range(2)]
    didx = [dst[h * e2:(h + 1) * e2].reshape(NW, nch, ch) for h in range(2)]
    zrows = jnp.zeros((n_pad // NS, OW), jnp.float32)

    nodes = x
    for s in range(s_steps):
        ws_cat = jnp.concatenate([eW0[s][:d], eWg0[s][:d]], axis=1)
        wd_cat = jnp.concatenate([eW0[s][d:2 * d], eWg0[s][d:2 * d]], axis=1)
        wa = jnp.zeros((16, 4 * h), jnp.float32)
        wa = wa.at[:9, :h].set(eW0[s][2 * d:2 * d + 9]).at[9, :h].set(eb0[s])
        wa = wa.at[:9, h:2 * h].set(eWg0[s][2 * d:2 * d + 9]).at[9, h:2 * h].set(ebg0[s])
        wa = wa.at[:9, 2 * h:3 * h].set(eW1[s][h:h + 9]).at[9, 2 * h:3 * h].set(eb1[s])
        wa = wa.at[:9, 3 * h:].set(eWg1[s][h:h + 9]).at[9, 3 * h:].set(ebg1[s])
        w1m = jnp.concatenate([eW1[s][:h], eWg1[s][:h]], axis=1)
        wh = jnp.stack([jnp.concatenate([nW[s, l][:d], nWg[s, l][:d]], axis=1)
                        for l in range(2)])
        wm = jnp.stack([
            jnp.zeros((OW, 2 * d), jnp.float32)
            .at[:41, :d].set(nW[s, l][d:d + 41])
            .at[:41, d:].set(nWg[s, l][d:d + 41])
            for l in range(2)])
        bc = jnp.stack([jnp.concatenate([nb[s, l], nbg[s, l]])[None, :]
                        for l in range(2)])

        stab, dtab = _tc_proj(nodes, ws_cat, wd_cat)
        rs0 = _sc_gather(stab, dtab, sidx[0], didx[0], e2, ch, nch, epw)
        rs1 = _sc_gather(stab, dtab, sidx[1], didx[1], e2, ch, nch, epw)
        ev0 = _tc_edge(rs0, wa, w1m)
        pa0 = _sc_scatter(ev0, didx[0], zrows, n_pad, ch, nch, epw)
        ev1 = _tc_edge(rs1, wa, w1m)
        pa1 = _sc_scatter(ev1, didx[1], zrows, n_pad, ch, nch, epw)
        nodes = _tc_node(pa0[0, :n], pa0[1, :n], pa1[0, :n], pa1[1, :n],
                         nodes, wh, wm, bc, lW[s], lb[s][None, :])
    return nodes


# speedup vs baseline: 8.3366x; 1.0204x over previous
import functools

import jax
import jax.numpy as jnp
import numpy as np
from jax import lax
from jax.experimental import pallas as pl
from jax.experimental.pallas import tpu as pltpu
from jax.experimental.pallas import tpu_sc as plsc

SQ3 = float(np.sqrt(3.0))
SQ15 = float(np.sqrt(15.0))
SQ5 = float(np.sqrt(5.0))

NC = 2
NS = 16
NW = NC * NS

TW = 128
OW = 128

NBLK = 1000
EBLK = 4000


def _proj_body(x_ref, ws_ref, wd_ref, s_ref, d_ref):
    xb = x_ref[...]
    xyz = xb[:, 0:3]
    zpad = jnp.zeros((xb.shape[0], TW - 67), jnp.float32)
    ps = jnp.dot(xb, ws_ref[...], preferred_element_type=jnp.float32)
    pd = jnp.dot(xb, wd_ref[...], preferred_element_type=jnp.float32)
    s_ref[...] = jnp.concatenate([ps, xyz, zpad], axis=1)
    d_ref[...] = jnp.concatenate([pd, -xyz, zpad], axis=1)


def _tc_proj(nodes, ws_cat, wd_cat):
    n = nodes.shape[0]
    grid = n // NBLK
    return pl.pallas_call(
        _proj_body,
        grid=(grid,),
        in_specs=[
            pl.BlockSpec((NBLK, 128), lambda i: (i, 0)),
            pl.BlockSpec((128, 64), lambda i: (0, 0)),
            pl.BlockSpec((128, 64), lambda i: (0, 0)),
        ],
        out_specs=[
            pl.BlockSpec((NBLK, TW), lambda i: (i, 0)),
            pl.BlockSpec((NBLK, TW), lambda i: (i, 0)),
        ],
        out_shape=[
            jax.ShapeDtypeStruct((n, TW), jnp.float32),
            jax.ShapeDtypeStruct((n, TW), jnp.float32),
        ],
        compiler_params=pltpu.CompilerParams(dimension_semantics=("parallel",)),
    )(nodes, ws_cat, wd_cat)


def _edge_body(rs_ref, wa_ref, w1m_ref, o_ref):
    rs = rs_ref[...]
    b = rs.shape[0]
    xt = rs[:, 64:80].T
    x, y, z = xt[0:1], xt[1:2], xt[2:3]
    n2 = x * x + y * y + z * z + 1e-12
    rinv = jax.lax.rsqrt(n2)
    ux, uy, uz = x * rinv, y * rinv, z * rinv
    one = jnp.ones((1, b), jnp.float32)
    at = jnp.concatenate([
        one, SQ3 * ux, SQ3 * uy, SQ3 * uz,
        SQ15 * ux * uy, SQ15 * uy * uz, (SQ5 / 2.0) * (3.0 * uz * uz - 1.0),
        SQ15 * ux * uz, (SQ15 / 2.0) * (ux * ux - uy * uy),
        one, jnp.zeros((6, b), jnp.float32)], axis=0)
    a16 = at.T
    av = jnp.dot(a16, wa_ref[...], preferred_element_type=jnp.float32)
    z0 = rs[:, :64] + av[:, :64]
    m0 = z0[:, :32] * jax.nn.sigmoid(z0[:, 32:64])
    z1 = (jnp.dot(m0, w1m_ref[...], preferred_element_type=jnp.float32)
          + av[:, 64:128])
    m1 = z1[:, :32] * jax.nn.sigmoid(z1[:, 32:64])
    o_ref[...] = jnp.concatenate(
        [m1, a16, jnp.zeros((b, OW - 48), jnp.float32)], axis=1)


def _tc_edge(rsrows, wa, w1m):
    e = rsrows.shape[0]
    grid = e // EBLK
    return pl.pallas_call(
        _edge_body,
        grid=(grid,),
        in_specs=[
            pl.BlockSpec((EBLK, TW), lambda i: (i, 0)),
            pl.BlockSpec((16, 128), lambda i: (0, 0)),
            pl.BlockSpec((32, 64), lambda i: (0, 0)),
        ],
        out_specs=pl.BlockSpec((EBLK, OW), lambda i: (i, 0)),
        out_shape=jax.ShapeDtypeStruct((e, OW), jnp.float32),
        compiler_params=pltpu.CompilerParams(dimension_semantics=("parallel",)),
    )(rsrows, wa, w1m)


def _node_body(p0_ref, p1_ref, p2_ref, p3_ref, x_ref, wh_ref, wm_ref, bc_ref,
               lw_ref, lb_ref, o_ref):
    acc = (p0_ref[...] + p1_ref[...]) + (p2_ref[...] + p3_ref[...])
    cnt = acc[:, 41:42]
    inv = 1.0 / jnp.maximum(cnt, 1.0)
    ms = acc * inv
    xb = x_ref[...]
    h = xb
    for l in range(2):
        p = (jnp.dot(h, wh_ref[l], preferred_element_type=jnp.float32)
             + jnp.dot(ms, wm_ref[l], preferred_element_type=jnp.float32)
             + bc_ref[l])
        h = p[:, :128] * jax.nn.sigmoid(p[:, 128:])
    o_ref[...] = (jnp.dot(h, lw_ref[...], preferred_element_type=jnp.float32)
                  + lb_ref[...] + xb)


def _tc_node(p0, p1, p2, p3, nodes, wh, wm, bc, lw, lb):
    n = nodes.shape[0]
    grid = n // NBLK
    return pl.pallas_call(
        _node_body,
        grid=(grid,),
        in_specs=[
            pl.BlockSpec((NBLK, OW), lambda i: (i, 0)),
            pl.BlockSpec((NBLK, OW), lambda i: (i, 0)),
            pl.BlockSpec((NBLK, OW), lambda i: (i, 0)),
            pl.BlockSpec((NBLK, OW), lambda i: (i, 0)),
            pl.BlockSpec((NBLK, 128), lambda i: (i, 0)),
            pl.BlockSpec((2, 128, 256), lambda i: (0, 0, 0)),
            pl.BlockSpec((2, OW, 256), lambda i: (0, 0, 0)),
            pl.BlockSpec((2, 1, 256), lambda i: (0, 0, 0)),
            pl.BlockSpec((128, 128), lambda i: (0, 0)),
            pl.BlockSpec((1, 128), lambda i: (0, 0)),
        ],
        out_specs=pl.BlockSpec((NBLK, 128), lambda i: (i, 0)),
        out_shape=jax.ShapeDtypeStruct((n, 128), jnp.float32),
        compiler_params=pltpu.CompilerParams(dimension_semantics=("parallel",)),
    )(p0, p1, p2, p3, nodes, wh, wm, bc, lw, lb)


def _sc_gather(stab, dtab, sidx3, didx3, e, ch, nch, epw):
    mesh = plsc.VectorSubcoreMesh(core_axis_name="c", subcore_axis_name="s",
                                  num_cores=NC, num_subcores=NS)

    @functools.partial(
        pl.kernel,
        out_type=jax.ShapeDtypeStruct((e, TW), jnp.float32),
        mesh=mesh,
        scratch_types=[
            pltpu.VMEM((nch, ch), jnp.int32),
            pltpu.VMEM((nch, ch), jnp.int32),
            pltpu.VMEM((ch, TW), jnp.float32),
            pltpu.VMEM((ch, TW), jnp.float32),
            pltpu.VMEM((ch, TW), jnp.float32),
            pltpu.VMEM((ch, TW), jnp.float32),
            pltpu.SemaphoreType.DMA,
            pltpu.SemaphoreType.DMA,
            pltpu.SemaphoreType.DMA,
            pltpu.SemaphoreType.DMA,
        ],
    )
    def k(stab_h, dtab_h, sidx_h, didx_h, out_h,
          sidx_v, didx_v, sA, dA, sB, dB, semsA, semdA, semsB, semdB):
        wid = lax.axis_index("s") * NC + lax.axis_index("c")
        base = wid * epw
        pltpu.sync_copy(sidx_h.at[wid], sidx_v)
        pltpu.sync_copy(didx_h.at[wid], didx_v)

        def issue(j, sbuf, dbuf, sem_s, sem_d):
            pltpu.async_copy(stab_h.at[sidx_v.at[j]], sbuf, sem_s)
            pltpu.async_copy(dtab_h.at[didx_v.at[j]], dbuf, sem_d)

        def wait(sbuf, dbuf, sem_s, sem_d):
            pltpu.make_async_copy(stab_h.at[sidx_v.at[0]], sbuf, sem_s).wait()
            pltpu.make_async_copy(dtab_h.at[didx_v.at[0]], dbuf, sem_d).wait()

        def process(j, sbuf, dbuf):
            def row(r, carry):
                for g in range(TW // 16):
                    sl = pl.ds(g * 16, 16)
                    sbuf[r, sl] = sbuf[r, sl] + dbuf[r, sl]
                return carry

            lax.fori_loop(0, ch, row, 0)
            pltpu.sync_copy(sbuf, out_h.at[pl.ds(base + j * ch, ch)])

        issue(0, sA, dA, semsA, semdA)

        def body(k2, carry):
            j = 2 * k2
            issue(j + 1, sB, dB, semsB, semdB)
            wait(sA, dA, semsA, semdA)
            process(j, sA, dA)
            issue(j + 2, sA, dA, semsA, semdA)
            wait(sB, dB, semsB, semdB)
            process(j + 1, sB, dB)
            return carry

        lax.fori_loop(0, (nch - 1) // 2, body, 0)
        wait(sA, dA, semsA, semdA)
        process(nch - 1, sA, dA)

    return k(stab, dtab, sidx3, didx3)


def _sc_scatter(vals, didx3, zrows, n_pad, ch, nch, epw):
    npt = n_pad // NS
    mesh = plsc.VectorSubcoreMesh(core_axis_name="c", subcore_axis_name="s",
                                  num_cores=NC, num_subcores=NS)

    @functools.partial(
        pl.kernel,
        out_type=jax.ShapeDtypeStruct((NC, n_pad, OW), jnp.float32),
        mesh=mesh,
        scratch_types=[
            pltpu.VMEM((nch, ch), jnp.int32),
            pltpu.VMEM((ch, OW), jnp.float32),
            pltpu.VMEM((ch, OW), jnp.float32),
            pltpu.SemaphoreType.DMA,
            pltpu.SemaphoreType.DMA,
            pltpu.VMEM_SHARED((n_pad, OW), jnp.float32),
        ],
    )
    def k(vals_h, didx_h, z_h, out_h, idx_v, bufA, bufB, semA, semB, acc_sh):
        c = lax.axis_index("c")
        s = lax.axis_index("s")
        wid = s * NC + c
        base = wid * epw
        pltpu.sync_copy(z_h, acc_sh.at[pl.ds(s * npt, npt)])
        pltpu.sync_copy(didx_h.at[wid], idx_v)
        plsc.subcore_barrier()

        def issue(j, buf, sem):
            pltpu.async_copy(vals_h.at[pl.ds(base + j * ch, ch)], buf, sem)

        def wait(buf, sem):
            pltpu.make_async_copy(vals_h.at[pl.ds(base, ch)], buf, sem).wait()

        def scat(j, buf):
            pltpu.sync_copy(buf, acc_sh.at[idx_v.at[j]], add=True)

        issue(0, bufA, semA)

        def body(k2, carry):
            j = 2 * k2
            issue(j + 1, bufB, semB)
            wait(bufA, semA)
            scat(j, bufA)
            issue(j + 2, bufA, semA)
            wait(bufB, semB)
            scat(j + 1, bufB)
            return carry

        lax.fori_loop(0, (nch - 1) // 2, body, 0)
        wait(bufA, semA)
        scat(nch - 1, bufA)
        plsc.subcore_barrier()
        pltpu.sync_copy(acc_sh.at[pl.ds(s * npt, npt)], out_h.at[c, pl.ds(s * npt, npt)])

    return k(vals, didx3, zrows)


def kernel(x, edge_index, eW0, eb0, eWg0, ebg0, eW1, eb1, eWg1, ebg1,
           nW, nb, nWg, nbg, lW, lb):
    n, d = x.shape
    e = edge_index.shape[1]
    s_steps = eW0.shape[0]
    h = eW0.shape[2]

    e2 = e // 2
    epw = e2 // NW
    ch = 40
    nch = epw // ch
    n_pad = ((n + 8 * NS - 1) // (8 * NS)) * (8 * NS)

    src = edge_index[0]
    dst = edge_index[1]
    sidx = [src[h * e2:(h + 1) * e2].reshape(NW, nch, ch) for h in range(2)]
    didx = [dst[h * e2:(h + 1) * e2].reshape(NW, nch, ch) for h in range(2)]
    zrows = jnp.zeros((n_pad // NS, OW), jnp.float32)

    nodes = x
    for s in range(s_steps):
        ws_cat = jnp.concatenate([eW0[s][:d], eWg0[s][:d]], axis=1)
        wd_cat = jnp.concatenate([eW0[s][d:2 * d], eWg0[s][d:2 * d]], axis=1)
        wa = jnp.zeros((16, 4 * h), jnp.float32)
        wa = wa.at[:9, :h].set(eW0[s][2 * d:2 * d + 9]).at[9, :h].set(eb0[s])
        wa = wa.at[:9, h:2 * h].set(eWg0[s][2 * d:2 * d + 9]).at[9, h:2 * h].set(ebg0[s])
        wa = wa.at[:9, 2 * h:3 * h].set(eW1[s][h:h + 9]).at[9, 2 * h:3 * h].set(eb1[s])
        wa = wa.at[:9, 3 * h:].set(eWg1[s][h:h + 9]).at[9, 3 * h:].set(ebg1[s])
        w1m = jnp.concatenate([eW1[s][:h], eWg1[s][:h]], axis=1)
        wh = jnp.stack([jnp.concatenate([nW[s, l][:d], nWg[s, l][:d]], axis=1)
                        for l in range(2)])
        wm = jnp.stack([
            jnp.zeros((OW, 2 * d), jnp.float32)
            .at[:41, :d].set(nW[s, l][d:d + 41])
            .at[:41, d:].set(nWg[s, l][d:d + 41])
            for l in range(2)])
        bc = jnp.stack([jnp.concatenate([nb[s, l], nbg[s, l]])[None, :]
                        for l in range(2)])

        stab, dtab = _tc_proj(nodes, ws_cat, wd_cat)
        rs0 = _sc_gather(stab, dtab, sidx[0], didx[0], e2, ch, nch, epw)
        rs1 = _sc_gather(stab, dtab, sidx[1], didx[1], e2, ch, nch, epw)
        ev0 = _tc_edge(rs0, wa, w1m)
        pa0 = _sc_scatter(ev0, didx[0], zrows, n_pad, ch, nch, epw)
        ev1 = _tc_edge(rs1, wa, w1m)
        pa1 = _sc_scatter(ev1, didx[1], zrows, n_pad, ch, nch, epw)
        nodes = _tc_node(pa0[0, :n], pa0[1, :n], pa1[0, :n], pa1[1, :n],
                         nodes, wh, wm, bc, lW[s], lb[s][None, :])
    return nodes
